# baseline (speedup 1.0000x reference)
import jax
import jax.numpy as jnp
from jax.experimental import pallas as pl

_TOPK = 8
_NCOL = 64
_NROW = 32768
_BLOCK = 1024


def _router_block(x_ref, epst_ref, out_ref, idx_ref):
    x = x_ref[...].T
    eps = epst_ref[...]
    noisy = x + eps * jax.nn.softplus(x)
    rows = jax.lax.broadcasted_iota(jnp.int32, noisy.shape, 0).astype(
        jnp.float32)
    work = noisy
    vals = []
    idxs = []
    for _ in range(_TOPK):
        m = jnp.max(work, axis=0, keepdims=True)
        sel = jnp.min(jnp.where(work == m, rows, float(_NCOL)), axis=0,
                      keepdims=True)
        vals.append(m)
        idxs.append(sel)
        work = jnp.where(rows == sel, -jnp.inf, work)
    v = jnp.concatenate(vals, axis=0)
    fi = jnp.concatenate(idxs, axis=0)
    p = jnp.exp(v - v[0:1])
    p = p / jnp.sum(p, axis=0, keepdims=True)
    out = jnp.zeros_like(x)
    for k in range(_TOPK):
        out = jnp.where(rows == fi[k : k + 1], p[k : k + 1], out)
    out_ref[...] = out.T
    idx_ref[...] = fi.astype(jnp.int32).T


def kernel(logits):
    eps_t = jax.random.normal(
        jax.random.key(42), logits.shape, dtype=logits.dtype
    ).T
    grid = (_NROW // _BLOCK,)
    router, indices = pl.pallas_call(
        _router_block,
        grid=grid,
        in_specs=[
            pl.BlockSpec((_BLOCK, _NCOL), lambda i: (i, 0)),
            pl.BlockSpec((_NCOL, _BLOCK), lambda i: (0, i)),
        ],
        out_specs=[
            pl.BlockSpec((_BLOCK, _NCOL), lambda i: (i, 0)),
            pl.BlockSpec((_BLOCK, _TOPK), lambda i: (i, 0)),
        ],
        out_shape=[
            jax.ShapeDtypeStruct((_NROW, _NCOL), logits.dtype),
            jax.ShapeDtypeStruct((_NROW, _TOPK), jnp.int32),
        ],
    )(logits, eps_t)
    return router, indices
